# baseline (speedup 1.0000x reference)
import functools

import jax
import jax.numpy as jnp
from jax import lax
from jax.experimental import pallas as pl
from jax.experimental.pallas import tpu as pltpu
from jax.experimental.pallas import tpu_sc as plsc

HEADS = 4
PE_DIM = 64
LANES = 16
NUM_CORES = 2
NUM_SUBCORES = 16
NUM_WORKERS = NUM_CORES * NUM_SUBCORES


def _sc_parent_gather(pjpe, parents_flat, bs, n_joints, rows_pad, rows_per_worker):
  mesh = plsc.VectorSubcoreMesh(
      core_axis_name="c",
      subcore_axis_name="s",
      num_cores=NUM_CORES,
      num_subcores=NUM_SUBCORES,
  )

  @functools.partial(
      pl.kernel,
      mesh=mesh,
      compiler_params=pltpu.CompilerParams(
          use_tc_tiling_on_sc=False, needs_layout_passes=False
      ),
      out_type=jax.ShapeDtypeStruct((rows_pad, PE_DIM), jnp.float32),
      scratch_types=[
          pltpu.VMEM((bs * n_joints,), jnp.int32),
          pltpu.VMEM((rows_per_worker,), jnp.int32),
          pltpu.VMEM((rows_per_worker, PE_DIM), jnp.float32),
          pltpu.SemaphoreType.DMA,
      ],
  )
  def gather_kernel(table_hbm, parents_hbm, out_hbm, par_v, idx_v, rows_v, sem):
    pltpu.sync_copy(parents_hbm, par_v)
    wid = lax.axis_index("s") * NUM_CORES + lax.axis_index("c")
    base = wid * rows_per_worker
    zero_row = table_hbm.shape[0] - 1
    for c in range(rows_per_worker // LANES):
      r = base + c * LANES + lax.iota(jnp.int32, LANES)
      j = lax.shift_right_logical(r, 4)
      b = lax.bitwise_and(r, 15)
      valid = j < n_joints
      flat = jnp.where(valid, b * n_joints + j, 0)
      pv = plsc.load_gather(par_v, [flat])
      idx_v[pl.ds(c * LANES, LANES)] = jnp.where(valid, pv, zero_row)
    pltpu.async_copy(table_hbm.at[idx_v], rows_v, sem).wait()
    pltpu.sync_copy(rows_v, out_hbm.at[pl.ds(base, rows_per_worker)])

  return gather_kernel(pjpe, parents_flat)


def _tc_add_body(x_ref, pjpe_ref, pp_ref, out_ref, tab_ref):
  n_joints, bs = x_ref.shape[1], x_ref.shape[2]

  @pl.when(pl.program_id(0) == 0)
  def _build_table():
    jrow = pjpe_ref[:n_joints]
    jb = jnp.broadcast_to(jrow[:, None, :], (n_joints, bs, PE_DIM))
    half = jnp.concatenate([jb, pp_ref[:n_joints]], axis=-1)
    tab_ref[...] = jnp.concatenate([half] * HEADS, axis=-1)

  out_ref[...] = x_ref[...] + tab_ref[...][None]


def kernel(x, parents, pjpe):
  frame_num, bs, n_joints, d_model = x.shape
  rows = n_joints * bs
  rows_per_worker = -(-rows // NUM_WORKERS)
  rows_per_worker = -(-rows_per_worker // LANES) * LANES
  rows_pad = rows_per_worker * NUM_WORKERS
  j_pad = rows_pad // bs

  pp = _sc_parent_gather(
      pjpe, parents.astype(jnp.int32).reshape(-1), bs, n_joints,
      rows_pad, rows_per_worker,
  )
  pp = pp.reshape(j_pad, bs, PE_DIM)

  xt = jnp.transpose(x, (0, 2, 1, 3))

  f_blk = 2
  out_t = pl.pallas_call(
      _tc_add_body,
      grid=(frame_num // f_blk,),
      in_specs=[
          pl.BlockSpec((f_blk, n_joints, bs, d_model), lambda f: (f, 0, 0, 0)),
          pl.BlockSpec(pjpe.shape, lambda f: (0, 0)),
          pl.BlockSpec((j_pad, bs, PE_DIM), lambda f: (0, 0, 0)),
      ],
      out_specs=pl.BlockSpec((f_blk, n_joints, bs, d_model), lambda f: (f, 0, 0, 0)),
      out_shape=jax.ShapeDtypeStruct(xt.shape, x.dtype),
      scratch_shapes=[pltpu.VMEM((n_joints, bs, d_model), jnp.float32)],
  )(xt, pjpe, pp)
  return jnp.transpose(out_t, (0, 2, 1, 3))


# speedup vs baseline: 1.0007x; 1.0007x over previous
import functools

import jax
import jax.numpy as jnp
from jax import lax
from jax.experimental import pallas as pl
from jax.experimental.pallas import tpu as pltpu
from jax.experimental.pallas import tpu_sc as plsc

HEADS = 4
PE_DIM = 64
LANES = 16
NUM_CORES = 2
NUM_SUBCORES = 16
NUM_WORKERS = NUM_CORES * NUM_SUBCORES


def _sc_parent_gather(pjpe, parents_flat, bs, n_joints, rows_pad, rows_per_worker):
  mesh = plsc.VectorSubcoreMesh(
      core_axis_name="c",
      subcore_axis_name="s",
      num_cores=NUM_CORES,
      num_subcores=NUM_SUBCORES,
  )

  @functools.partial(
      pl.kernel,
      mesh=mesh,
      compiler_params=pltpu.CompilerParams(
          use_tc_tiling_on_sc=False,
          needs_layout_passes=False,
          skip_device_barrier=True,
      ),
      out_type=jax.ShapeDtypeStruct((rows_pad, PE_DIM), jnp.float32),
      scratch_types=[
          pltpu.VMEM((bs * n_joints,), jnp.int32),
          pltpu.VMEM((rows_per_worker,), jnp.int32),
          pltpu.VMEM((rows_per_worker, PE_DIM), jnp.float32),
          pltpu.SemaphoreType.DMA,
      ],
  )
  def gather_kernel(table_hbm, parents_hbm, out_hbm, par_v, idx_v, rows_v, sem):
    pltpu.sync_copy(parents_hbm, par_v)
    wid = lax.axis_index("s") * NUM_CORES + lax.axis_index("c")
    base = wid * rows_per_worker
    zero_row = table_hbm.shape[0] - 1
    for c in range(rows_per_worker // LANES):
      r = base + c * LANES + lax.iota(jnp.int32, LANES)
      j = lax.shift_right_logical(r, 4)
      b = lax.bitwise_and(r, 15)
      valid = j < n_joints
      flat = jnp.where(valid, b * n_joints + j, 0)
      pv = plsc.load_gather(par_v, [flat])
      idx_v[pl.ds(c * LANES, LANES)] = jnp.where(valid, pv, zero_row)
    pltpu.async_copy(table_hbm.at[idx_v], rows_v, sem).wait()
    pltpu.sync_copy(rows_v, out_hbm.at[pl.ds(base, rows_per_worker)])

  return gather_kernel(pjpe, parents_flat)


def _tc_add_body(x_ref, pjpe_ref, pp_ref, out_ref, tab_ref):
  n_joints, bs = x_ref.shape[1], x_ref.shape[2]

  @pl.when(pl.program_id(0) == 0)
  def _build_table():
    jrow = pjpe_ref[:n_joints]
    jb = jnp.broadcast_to(jrow[:, None, :], (n_joints, bs, PE_DIM))
    half = jnp.concatenate([jb, pp_ref[:n_joints]], axis=-1)
    tab_ref[...] = jnp.concatenate([half] * HEADS, axis=-1)

  out_ref[...] = x_ref[...] + tab_ref[...][None]


def kernel(x, parents, pjpe):
  frame_num, bs, n_joints, d_model = x.shape
  rows = n_joints * bs
  rows_per_worker = -(-rows // NUM_WORKERS)
  rows_per_worker = -(-rows_per_worker // LANES) * LANES
  rows_pad = rows_per_worker * NUM_WORKERS
  j_pad = rows_pad // bs

  pp = _sc_parent_gather(
      pjpe, parents.astype(jnp.int32).reshape(-1), bs, n_joints,
      rows_pad, rows_per_worker,
  )
  pp = pp.reshape(j_pad, bs, PE_DIM)

  xt = jnp.transpose(x, (0, 2, 1, 3))

  f_blk = 2
  out_t = pl.pallas_call(
      _tc_add_body,
      grid=(frame_num // f_blk,),
      in_specs=[
          pl.BlockSpec((f_blk, n_joints, bs, d_model), lambda f: (f, 0, 0, 0)),
          pl.BlockSpec(pjpe.shape, lambda f: (0, 0)),
          pl.BlockSpec((j_pad, bs, PE_DIM), lambda f: (0, 0, 0)),
      ],
      out_specs=pl.BlockSpec((f_blk, n_joints, bs, d_model), lambda f: (f, 0, 0, 0)),
      out_shape=jax.ShapeDtypeStruct(xt.shape, x.dtype),
      scratch_shapes=[pltpu.VMEM((n_joints, bs, d_model), jnp.float32)],
      compiler_params=pltpu.CompilerParams(skip_device_barrier=True),
  )(xt, pjpe, pp)
  return jnp.transpose(out_t, (0, 2, 1, 3))


# speedup vs baseline: 1.0374x; 1.0367x over previous
import functools

import jax
import jax.numpy as jnp
from jax import lax
from jax.experimental import pallas as pl
from jax.experimental.pallas import tpu as pltpu
from jax.experimental.pallas import tpu_sc as plsc

HEADS = 4
PE_DIM = 64
LANE_PAD = 128
NUM_CORES = 2
NUM_SUBCORES = 16
NUM_WORKERS = NUM_CORES * NUM_SUBCORES


def _sc_parent_gather(pjpe128, pidx, rows_pad, rows_per_worker):
  mesh = plsc.VectorSubcoreMesh(
      core_axis_name="c",
      subcore_axis_name="s",
      num_cores=NUM_CORES,
      num_subcores=NUM_SUBCORES,
  )

  @functools.partial(
      pl.kernel,
      mesh=mesh,
      out_type=jax.ShapeDtypeStruct((rows_pad, LANE_PAD), jnp.float32),
      scratch_types=[
          pltpu.VMEM((rows_per_worker,), jnp.int32),
          pltpu.VMEM((rows_per_worker, LANE_PAD), jnp.float32),
          pltpu.SemaphoreType.DMA,
      ],
  )
  def gather_kernel(table_hbm, pidx_hbm, out_hbm, idx_v, rows_v, sem):
    wid = lax.axis_index("s") * NUM_CORES + lax.axis_index("c")
    base = wid * rows_per_worker
    pltpu.sync_copy(pidx_hbm.at[pl.ds(base, rows_per_worker)], idx_v)
    pltpu.async_copy(table_hbm.at[idx_v], rows_v, sem).wait()
    pltpu.sync_copy(rows_v, out_hbm.at[pl.ds(base, rows_per_worker)])

  return gather_kernel(pjpe128, pidx)


def _tc_add_body(x_ref, pjpe_ref, pp_ref, out_ref, tab_ref):
  n_joints, bs = x_ref.shape[1], x_ref.shape[2]

  @pl.when(pl.program_id(0) == 0)
  def _build_table():
    jrow = pjpe_ref[:n_joints]
    jb = jnp.broadcast_to(jrow[:, None, :], (n_joints, bs, PE_DIM))
    half = jnp.concatenate([jb, pp_ref[:n_joints, :, :PE_DIM]], axis=-1)
    tab_ref[...] = jnp.concatenate([half] * HEADS, axis=-1)

  out_ref[...] = x_ref[...] + tab_ref[...][None]


def kernel(x, parents, pjpe):
  frame_num, bs, n_joints, d_model = x.shape
  zero_row = pjpe.shape[0] - 1
  j_pad = -(-(n_joints + 1) // 8) * 8
  rows_pad = j_pad * bs
  rows_per_worker = rows_pad // NUM_WORKERS
  assert rows_per_worker % 8 == 0

  pidx = jnp.concatenate(
      [parents.astype(jnp.int32).T,
       jnp.full((j_pad - n_joints, bs), zero_row, jnp.int32)],
      axis=0,
  ).reshape(-1)
  pjpe128 = jnp.pad(pjpe, ((0, 0), (0, LANE_PAD - PE_DIM)))

  pp = _sc_parent_gather(pjpe128, pidx, rows_pad, rows_per_worker)
  pp = pp.reshape(j_pad, bs, LANE_PAD)

  xt = jnp.transpose(x, (0, 2, 1, 3))

  f_blk = 2
  out_t = pl.pallas_call(
      _tc_add_body,
      grid=(frame_num // f_blk,),
      in_specs=[
          pl.BlockSpec((f_blk, n_joints, bs, d_model), lambda f: (f, 0, 0, 0)),
          pl.BlockSpec(pjpe.shape, lambda f: (0, 0)),
          pl.BlockSpec((j_pad, bs, LANE_PAD), lambda f: (0, 0, 0)),
      ],
      out_specs=pl.BlockSpec((f_blk, n_joints, bs, d_model), lambda f: (f, 0, 0, 0)),
      out_shape=jax.ShapeDtypeStruct(xt.shape, x.dtype),
      scratch_shapes=[pltpu.VMEM((n_joints, bs, d_model), jnp.float32)],
  )(xt, pjpe, pp)
  return jnp.transpose(out_t, (0, 2, 1, 3))


# speedup vs baseline: 1.0381x; 1.0006x over previous
import functools

import jax
import jax.numpy as jnp
from jax import lax
from jax.experimental import pallas as pl
from jax.experimental.pallas import tpu as pltpu
from jax.experimental.pallas import tpu_sc as plsc

HEADS = 4
PE_DIM = 64
LANE_PAD = 128
NUM_CORES = 2
NUM_SUBCORES = 16
NUM_WORKERS = NUM_CORES * NUM_SUBCORES


def _sc_parent_gather(pjpe128, pidx, rows_pad, rows_per_worker):
  mesh = plsc.VectorSubcoreMesh(
      core_axis_name="c",
      subcore_axis_name="s",
      num_cores=NUM_CORES,
      num_subcores=NUM_SUBCORES,
  )

  @functools.partial(
      pl.kernel,
      mesh=mesh,
      out_type=jax.ShapeDtypeStruct((rows_pad, LANE_PAD), jnp.float32),
      scratch_types=[
          pltpu.VMEM((rows_per_worker,), jnp.int32),
          pltpu.VMEM((rows_per_worker, LANE_PAD), jnp.float32),
          pltpu.SemaphoreType.DMA,
      ],
  )
  def gather_kernel(table_hbm, pidx_hbm, out_hbm, idx_v, rows_v, sem):
    wid = lax.axis_index("s") * NUM_CORES + lax.axis_index("c")
    base = wid * rows_per_worker
    pltpu.sync_copy(pidx_hbm.at[pl.ds(base, rows_per_worker)], idx_v)
    pltpu.async_copy(table_hbm.at[idx_v], rows_v, sem).wait()
    pltpu.sync_copy(rows_v, out_hbm.at[pl.ds(base, rows_per_worker)])

  return gather_kernel(pjpe128, pidx)


def _tc_add_body(x_ref, pjpe_ref, pp_ref, out_ref, tab_ref):
  n_joints, bs = x_ref.shape[1], x_ref.shape[2]

  @pl.when(pl.program_id(0) == 0)
  def _build_table():
    jrow = pjpe_ref[:n_joints, :PE_DIM]
    jb = jnp.broadcast_to(jrow[:, None, :], (n_joints, bs, PE_DIM))
    half = jnp.concatenate([jb, pp_ref[:n_joints, :, :PE_DIM]], axis=-1)
    tab_ref[...] = jnp.concatenate([half] * HEADS, axis=-1)

  out_ref[...] = x_ref[...] + tab_ref[...][None]


def kernel(x, parents, pjpe):
  frame_num, bs, n_joints, d_model = x.shape
  zero_row = pjpe.shape[0] - 1
  j_pad = -(-(n_joints + 1) // 8) * 8
  rows_pad = j_pad * bs
  rows_per_worker = rows_pad // NUM_WORKERS
  assert rows_per_worker % 8 == 0

  pidx = jnp.concatenate(
      [parents.astype(jnp.int32).T,
       jnp.full((j_pad - n_joints, bs), zero_row, jnp.int32)],
      axis=0,
  ).reshape(-1)
  pjpe128 = jnp.pad(pjpe, ((0, 0), (0, LANE_PAD - PE_DIM)))

  pp = _sc_parent_gather(pjpe128, pidx, rows_pad, rows_per_worker)
  pp = pp.reshape(j_pad, bs, LANE_PAD)

  xt = jnp.transpose(x, (0, 2, 1, 3))

  f_blk = 2
  out_t = pl.pallas_call(
      _tc_add_body,
      grid=(frame_num // f_blk,),
      in_specs=[
          pl.BlockSpec((f_blk, n_joints, bs, d_model), lambda f: (f, 0, 0, 0)),
          pl.BlockSpec(pjpe128.shape, lambda f: (0, 0)),
          pl.BlockSpec((j_pad, bs, LANE_PAD), lambda f: (0, 0, 0)),
      ],
      out_specs=pl.BlockSpec((f_blk, n_joints, bs, d_model), lambda f: (f, 0, 0, 0)),
      out_shape=jax.ShapeDtypeStruct(xt.shape, x.dtype),
      scratch_shapes=[pltpu.VMEM((n_joints, bs, d_model), jnp.float32)],
  )(xt, pjpe128, pp)
  return jnp.transpose(out_t, (0, 2, 1, 3))


# speedup vs baseline: 1.0382x; 1.0001x over previous
import functools

import jax
import jax.numpy as jnp
from jax import lax
from jax.experimental import pallas as pl
from jax.experimental.pallas import tpu as pltpu
from jax.experimental.pallas import tpu_sc as plsc

HEADS = 4
PE_DIM = 64
LANE_PAD = 128
NUM_CORES = 2
NUM_SUBCORES = 16
NUM_WORKERS = NUM_CORES * NUM_SUBCORES


def _sc_parent_gather(pjpe128, pidx, rows_pad, rows_per_worker):
  mesh = plsc.VectorSubcoreMesh(
      core_axis_name="c",
      subcore_axis_name="s",
      num_cores=NUM_CORES,
      num_subcores=NUM_SUBCORES,
  )

  @functools.partial(
      pl.kernel,
      mesh=mesh,
      out_type=jax.ShapeDtypeStruct((rows_pad, LANE_PAD), jnp.float32),
      scratch_types=[
          pltpu.VMEM((rows_per_worker,), jnp.int32),
          pltpu.VMEM((rows_per_worker, LANE_PAD), jnp.float32),
          pltpu.SemaphoreType.DMA,
      ],
  )
  def gather_kernel(table_hbm, pidx_hbm, out_hbm, idx_v, rows_v, sem):
    wid = lax.axis_index("s") * NUM_CORES + lax.axis_index("c")
    base = wid * rows_per_worker
    pltpu.sync_copy(pidx_hbm.at[pl.ds(base, rows_per_worker)], idx_v)
    pltpu.async_copy(table_hbm.at[idx_v], rows_v, sem).wait()
    pltpu.sync_copy(rows_v, out_hbm.at[pl.ds(base, rows_per_worker)])

  return gather_kernel(pjpe128, pidx)


def _tc_add_body(x_ref, pjpe_ref, pp_ref, out_ref, tab_ref):
  n_joints, bs = x_ref.shape[1], x_ref.shape[2]

  @pl.when(pl.program_id(0) == 0)
  def _build_table():
    jrow = pjpe_ref[:n_joints, :PE_DIM]
    jb = jnp.broadcast_to(jrow[:, None, :], (n_joints, bs, PE_DIM))
    half = jnp.concatenate([jb, pp_ref[:n_joints, :, :PE_DIM]], axis=-1)
    tab_ref[...] = jnp.concatenate([half] * HEADS, axis=-1)

  out_ref[...] = x_ref[...] + tab_ref[...][None]


def kernel(x, parents, pjpe):
  frame_num, bs, n_joints, d_model = x.shape
  zero_row = pjpe.shape[0] - 1
  j_pad = -(-(n_joints + 1) // 8) * 8
  rows_pad = j_pad * bs
  rows_per_worker = rows_pad // NUM_WORKERS
  assert rows_per_worker % 8 == 0

  pidx = jnp.pad(
      parents.astype(jnp.int32), ((0, 0), (0, j_pad - n_joints)),
      constant_values=zero_row,
  ).T.reshape(-1)
  pjpe128 = jnp.pad(pjpe, ((0, 0), (0, LANE_PAD - PE_DIM)))

  pp = _sc_parent_gather(pjpe128, pidx, rows_pad, rows_per_worker)
  pp = pp.reshape(j_pad, bs, LANE_PAD)

  xt = jnp.transpose(x, (0, 2, 1, 3))

  f_blk = 2
  out_t = pl.pallas_call(
      _tc_add_body,
      grid=(frame_num // f_blk,),
      in_specs=[
          pl.BlockSpec((f_blk, n_joints, bs, d_model), lambda f: (f, 0, 0, 0)),
          pl.BlockSpec(pjpe128.shape, lambda f: (0, 0)),
          pl.BlockSpec((j_pad, bs, LANE_PAD), lambda f: (0, 0, 0)),
      ],
      out_specs=pl.BlockSpec((f_blk, n_joints, bs, d_model), lambda f: (f, 0, 0, 0)),
      out_shape=jax.ShapeDtypeStruct(xt.shape, x.dtype),
      scratch_shapes=[pltpu.VMEM((n_joints, bs, d_model), jnp.float32)],
  )(xt, pjpe128, pp)
  return jnp.transpose(out_t, (0, 2, 1, 3))
